# baseline (speedup 1.0000x reference)
import functools

import jax
import jax.numpy as jnp
from jax import lax
from jax.experimental import pallas as pl
from jax.experimental.pallas import tpu as pltpu
from jax.experimental.pallas import tpu_sc as plsc

N = 320000
D = 128
NSYS = 10000
NC = 2
NS = 16
CHUNK = 128
IROWS = CHUNK // 128
NCHUNKS = N // CHUNK
CHUNKS_PER_SC = NCHUNKS // NC
PER_TILE = (CHUNKS_PER_SC // NS) & ~1
NBUF = 4
SEG_PER_SC = NSYS // NC
DUMMY = SEG_PER_SC
ACC_ROWS = SEG_PER_SC + 8
WB = 40
NWB = SEG_PER_SC // WB


def _sc_segment_sum(features, batch_index):
    mesh = plsc.VectorSubcoreMesh(core_axis_name="c", subcore_axis_name="s")

    @functools.partial(
        pl.kernel,
        out_type=jax.ShapeDtypeStruct((NSYS, D), jnp.float32),
        mesh=mesh,
        scratch_types=[
            pltpu.VMEM((NBUF, CHUNK, D), jnp.float32),
            pltpu.VMEM((NBUF, IROWS, 128), jnp.int32),
            pltpu.VMEM((WB, D), jnp.float32),
            pltpu.VMEM_SHARED((ACC_ROWS, D), jnp.float32),
            pltpu.SemaphoreType.DMA,
            pltpu.SemaphoreType.DMA,
            pltpu.SemaphoreType.DMA,
            pltpu.SemaphoreType.DMA,
            pltpu.SemaphoreType.DMA,
            pltpu.SemaphoreType.DMA,
            pltpu.SemaphoreType.DMA,
            pltpu.SemaphoreType.DMA,
        ],
    )
    def body(feat_hbm, idx_hbm, out_hbm, row_v, idx_v, zero_v, acc,
             g0, g1, g2, g3, s0, s1, s2, s3):
        c = lax.axis_index("c")
        t = lax.axis_index("s")
        segbase = SEG_PER_SC * c

        dummy_vec = jnp.full((16,), DUMMY, jnp.int32)

        def localize_idx(p, invalid_off):
            for q in range(IROWS):
                for k in range(128 // 16):
                    v = idx_v[p, q, pl.ds(16 * k, 16)] - segbase + invalid_off
                    m = (v >= 0) & (v < SEG_PER_SC)
                    idx_v[p, q, pl.ds(16 * k, 16)] = jnp.where(m, v, dummy_vec)

        base = CHUNKS_PER_SC * c
        gsem = (g0, g1, g2, g3)
        ssem = (s0, s1, s2, s3)

        def rstart(i):
            return CHUNK * (base + t + NS * i)

        def start_gather(i, p):
            r = rstart(i)
            for q in range(IROWS):
                pltpu.async_copy(
                    idx_hbm.at[pl.ds(r + 128 * q, 128)], idx_v.at[p, q], gsem[p]
                )
            pltpu.async_copy(feat_hbm.at[pl.ds(r, CHUNK)], row_v.at[p], gsem[p])

        def wait_gather(i, p):
            r = rstart(i)
            for q in range(IROWS):
                pltpu.make_async_copy(
                    idx_hbm.at[pl.ds(r + 128 * q, 128)], idx_v.at[p, q], gsem[p]
                ).wait()
            pltpu.make_async_copy(
                feat_hbm.at[pl.ds(r, CHUNK)], row_v.at[p], gsem[p]
            ).wait()

        def start_scatter(p):
            for q in range(IROWS):
                pltpu.async_copy(
                    row_v.at[p, pl.ds(128 * q, 128)],
                    acc.at[idx_v.at[p, q]],
                    ssem[p],
                    add=True,
                )

        def wait_scatter(p):
            for q in range(IROWS):
                pltpu.make_async_copy(
                    row_v.at[p, pl.ds(128 * q, 128)],
                    acc.at[idx_v.at[p, q]],
                    ssem[p],
                ).wait()

        def step(i, p, wait_prev, issue_gather):
            wait_gather(i, p)
            localize_idx(p, 0)
            if issue_gather:
                g = (p + 2) % NBUF
                if wait_prev:
                    wait_scatter(g)
                start_gather(i + 2, g)
            start_scatter(p)

        start_gather(0, 0)
        start_gather(1, 1)

        def zrow(i, _):
            for k in range(D // 16):
                zero_v[i, pl.ds(16 * k, 16)] = jnp.zeros((16,), jnp.float32)
            return 0

        lax.fori_loop(0, WB, zrow, 0)

        def zchunk(i, _):
            j = t + NS * i

            @pl.when(j < NWB)
            def _():
                pltpu.sync_copy(zero_v, acc.at[pl.ds(WB * j, WB)])

            return 0

        lax.fori_loop(0, (NWB + NS - 1) // NS, zchunk, 0)
        plsc.subcore_barrier()

        step(0, 0, False, True)
        step(1, 1, False, True)
        step(2, 2, True, True)
        step(3, 3, True, True)

        def aloop(i4, _):
            i = NBUF * i4
            for p in range(NBUF):
                step(i + p, p, True, True)
            return 0

        lax.fori_loop(1, (PER_TILE - 2) // NBUF, aloop, 0)

        step(PER_TILE - 2, (PER_TILE - 2) % NBUF, False, False)
        step(PER_TILE - 1, (PER_TILE - 1) % NBUF, False, False)
        for p in range(NBUF):
            wait_scatter(p)

        n_left = CHUNKS_PER_SC - NS * PER_TILE
        for e in range((n_left + NS - 1) // NS):

            @pl.when(NS * e + t < n_left)
            def _():
                r = CHUNK * (base + NS * PER_TILE + NS * e + t)
                for q in range(IROWS):
                    pltpu.sync_copy(
                        idx_hbm.at[pl.ds(r + 128 * q, 128)], idx_v.at[0, q]
                    )
                pltpu.sync_copy(feat_hbm.at[pl.ds(r, CHUNK)], row_v.at[0])
                localize_idx(0, 0)
                for q in range(IROWS):
                    pltpu.sync_copy(
                        row_v.at[0, pl.ds(128 * q, 128)],
                        acc.at[idx_v.at[0, q]],
                        add=True,
                    )

        k0 = (1 - c) * (CHUNKS_PER_SC + t) + c * (CHUNKS_PER_SC - 1 - t)
        kstep = NS - 2 * NS * c

        max_sweep = (CHUNKS_PER_SC + NS - 1) // NS

        def sweep_body(i, cont):
            k = k0 + kstep * i
            kc = jnp.clip(k, 0, NCHUNKS - 1)
            ok = (cont == 1) & (k >= 0) & (k < NCHUNKS)

            @pl.when(ok)
            def _():
                for q in range(IROWS):
                    pltpu.sync_copy(
                        idx_hbm.at[pl.ds(CHUNK * kc + 128 * q, 128)],
                        idx_v.at[0, q],
                    )

            mn = idx_v[0, 0, pl.ds(0, 16)][0]
            mx = idx_v[0, IROWS - 1, pl.ds(112, 16)][15]
            a = (mn < SEG_PER_SC).astype(jnp.int32)
            b = (mx >= SEG_PER_SC).astype(jnp.int32)
            has_own = (1 - c) * a + c * b
            proceed = ok & (has_own == 1)

            @pl.when(proceed)
            def _():
                pltpu.sync_copy(
                    feat_hbm.at[pl.ds(CHUNK * kc, CHUNK)], row_v.at[0]
                )
                localize_idx(0, 0)
                for q in range(IROWS):
                    pltpu.sync_copy(
                        row_v.at[0, pl.ds(128 * q, 128)],
                        acc.at[idx_v.at[0, q]],
                        add=True,
                    )

            return proceed.astype(jnp.int32)

        lax.fori_loop(0, max_sweep, sweep_body, jnp.int32(1))
        plsc.subcore_barrier()

        def wchunk(i, _):
            j = t + NS * i

            @pl.when(j < NWB)
            def _():
                pltpu.sync_copy(
                    acc.at[pl.ds(WB * j, WB)],
                    out_hbm.at[pl.ds(SEG_PER_SC * c + WB * j, WB)],
                )

            return 0

        lax.fori_loop(0, (NWB + NS - 1) // NS, wchunk, 0)

    return body(features, batch_index)


def kernel(features, batch_index, natoms):
    del natoms
    bi = batch_index.astype(jnp.int32)
    return _sc_segment_sum(features, bi)


# speedup vs baseline: 1.0310x; 1.0310x over previous
import functools

import jax
import jax.numpy as jnp
from jax import lax
from jax.experimental import pallas as pl
from jax.experimental.pallas import tpu as pltpu
from jax.experimental.pallas import tpu_sc as plsc

N = 320000
D = 128
NSYS = 10000
NC = 2
NS = 16
CHUNK = 128
IROWS = CHUNK // 128
NCHUNKS = N // CHUNK
CHUNKS_PER_SC = NCHUNKS // NC
PER_TILE = (CHUNKS_PER_SC // NS) & ~1
NBUF = 3
SEG_PER_SC = NSYS // NC
DUMMY = SEG_PER_SC
ACC_ROWS = SEG_PER_SC + 8
WB = 40
NWB = SEG_PER_SC // WB


def _sc_segment_sum(features, batch_index):
    mesh = plsc.VectorSubcoreMesh(core_axis_name="c", subcore_axis_name="s")

    @functools.partial(
        pl.kernel,
        out_type=jax.ShapeDtypeStruct((NSYS, D), jnp.float32),
        mesh=mesh,
        scratch_types=[
            pltpu.VMEM((NBUF, CHUNK, D), jnp.float32),
            pltpu.VMEM((NBUF, IROWS, 128), jnp.int32),
            pltpu.VMEM((WB, D), jnp.float32),
            pltpu.VMEM_SHARED((ACC_ROWS, D), jnp.float32),
            pltpu.SemaphoreType.DMA,
            pltpu.SemaphoreType.DMA,
            pltpu.SemaphoreType.DMA,
            pltpu.SemaphoreType.DMA,
            pltpu.SemaphoreType.DMA,
            pltpu.SemaphoreType.DMA,
        ],
    )
    def body(feat_hbm, idx_hbm, out_hbm, row_v, idx_v, zero_v, acc,
             g0, g1, g2, s0, s1, s2):
        c = lax.axis_index("c")
        t = lax.axis_index("s")
        segbase = SEG_PER_SC * c

        dummy_vec = jnp.full((16,), DUMMY, jnp.int32)

        def localize_idx(p, invalid_off):
            for q in range(IROWS):
                for k in range(128 // 16):
                    v = idx_v[p, q, pl.ds(16 * k, 16)] - segbase + invalid_off
                    m = (v >= 0) & (v < SEG_PER_SC)
                    idx_v[p, q, pl.ds(16 * k, 16)] = jnp.where(m, v, dummy_vec)

        base = CHUNKS_PER_SC * c
        gsem = (g0, g1, g2)
        ssem = (s0, s1, s2)

        def rstart(i):
            return CHUNK * (base + t + NS * i)

        def start_gather(i, p):
            r = rstart(i)
            for q in range(IROWS):
                pltpu.async_copy(
                    idx_hbm.at[pl.ds(r + 128 * q, 128)], idx_v.at[p, q], gsem[p]
                )
            pltpu.async_copy(feat_hbm.at[pl.ds(r, CHUNK)], row_v.at[p], gsem[p])

        def wait_gather(i, p):
            r = rstart(i)
            for q in range(IROWS):
                pltpu.make_async_copy(
                    idx_hbm.at[pl.ds(r + 128 * q, 128)], idx_v.at[p, q], gsem[p]
                ).wait()
            pltpu.make_async_copy(
                feat_hbm.at[pl.ds(r, CHUNK)], row_v.at[p], gsem[p]
            ).wait()

        def start_scatter(p):
            for q in range(IROWS):
                pltpu.async_copy(
                    row_v.at[p, pl.ds(128 * q, 128)],
                    acc.at[idx_v.at[p, q]],
                    ssem[p],
                    add=True,
                )

        def wait_scatter(p):
            for q in range(IROWS):
                pltpu.make_async_copy(
                    row_v.at[p, pl.ds(128 * q, 128)],
                    acc.at[idx_v.at[p, q]],
                    ssem[p],
                ).wait()

        def step(i, p, wait_prev, issue_gather):
            wait_gather(i, p)
            localize_idx(p, 0)
            if issue_gather:
                g = (p + 2) % NBUF
                if wait_prev:
                    wait_scatter(g)
                start_gather(i + 2, g)
            start_scatter(p)

        start_gather(0, 0)
        start_gather(1, 1)

        def zrow(i, _):
            for k in range(D // 16):
                zero_v[i, pl.ds(16 * k, 16)] = jnp.zeros((16,), jnp.float32)
            return 0

        lax.fori_loop(0, WB, zrow, 0)

        def zchunk(i, _):
            j = t + NS * i

            @pl.when(j < NWB)
            def _():
                pltpu.sync_copy(zero_v, acc.at[pl.ds(WB * j, WB)])

            return 0

        lax.fori_loop(0, (NWB + NS - 1) // NS, zchunk, 0)
        plsc.subcore_barrier()

        step(0, 0, False, True)
        step(1, 1, True, True)
        step(2, 2, True, True)

        def aloop(i4, _):
            i = NBUF * i4
            for p in range(NBUF):
                step(i + p, p, True, True)
            return 0

        lax.fori_loop(1, PER_TILE // NBUF - 1, aloop, 0)

        step(PER_TILE - 3, (PER_TILE - 3) % NBUF, True, True)
        step(PER_TILE - 2, (PER_TILE - 2) % NBUF, False, False)
        step(PER_TILE - 1, (PER_TILE - 1) % NBUF, False, False)
        for p in range(NBUF):
            wait_scatter(p)

        n_left = CHUNKS_PER_SC - NS * PER_TILE
        for e in range((n_left + NS - 1) // NS):

            @pl.when(NS * e + t < n_left)
            def _():
                r = CHUNK * (base + NS * PER_TILE + NS * e + t)
                for q in range(IROWS):
                    pltpu.sync_copy(
                        idx_hbm.at[pl.ds(r + 128 * q, 128)], idx_v.at[0, q]
                    )
                pltpu.sync_copy(feat_hbm.at[pl.ds(r, CHUNK)], row_v.at[0])
                localize_idx(0, 0)
                for q in range(IROWS):
                    pltpu.sync_copy(
                        row_v.at[0, pl.ds(128 * q, 128)],
                        acc.at[idx_v.at[0, q]],
                        add=True,
                    )

        k0 = (1 - c) * (CHUNKS_PER_SC + t) + c * (CHUNKS_PER_SC - 1 - t)
        kstep = NS - 2 * NS * c

        max_sweep = (CHUNKS_PER_SC + NS - 1) // NS

        def sweep_body(i, cont):
            k = k0 + kstep * i
            kc = jnp.clip(k, 0, NCHUNKS - 1)
            ok = (cont == 1) & (k >= 0) & (k < NCHUNKS)

            @pl.when(ok)
            def _():
                for q in range(IROWS):
                    pltpu.sync_copy(
                        idx_hbm.at[pl.ds(CHUNK * kc + 128 * q, 128)],
                        idx_v.at[0, q],
                    )

            mn = idx_v[0, 0, pl.ds(0, 16)][0]
            mx = idx_v[0, IROWS - 1, pl.ds(112, 16)][15]
            a = (mn < SEG_PER_SC).astype(jnp.int32)
            b = (mx >= SEG_PER_SC).astype(jnp.int32)
            has_own = (1 - c) * a + c * b
            proceed = ok & (has_own == 1)

            @pl.when(proceed)
            def _():
                pltpu.sync_copy(
                    feat_hbm.at[pl.ds(CHUNK * kc, CHUNK)], row_v.at[0]
                )
                localize_idx(0, 0)
                for q in range(IROWS):
                    pltpu.sync_copy(
                        row_v.at[0, pl.ds(128 * q, 128)],
                        acc.at[idx_v.at[0, q]],
                        add=True,
                    )

            return proceed.astype(jnp.int32)

        lax.fori_loop(0, max_sweep, sweep_body, jnp.int32(1))
        plsc.subcore_barrier()

        def wchunk(i, _):
            j = t + NS * i

            @pl.when(j < NWB)
            def _():
                pltpu.sync_copy(
                    acc.at[pl.ds(WB * j, WB)],
                    out_hbm.at[pl.ds(SEG_PER_SC * c + WB * j, WB)],
                )

            return 0

        lax.fori_loop(0, (NWB + NS - 1) // NS, wchunk, 0)

    return body(features, batch_index)


def kernel(features, batch_index, natoms):
    del natoms
    bi = batch_index.astype(jnp.int32)
    return _sc_segment_sum(features, bi)


# speedup vs baseline: 1.0563x; 1.0245x over previous
import functools

import jax
import jax.numpy as jnp
from jax import lax
from jax.experimental import pallas as pl
from jax.experimental.pallas import tpu as pltpu
from jax.experimental.pallas import tpu_sc as plsc

N = 320000
D = 128
NSYS = 10000
NC = 2
NS = 16
CHUNK = 128
IROWS = CHUNK // 128
NCHUNKS = N // CHUNK
CHUNKS_PER_SC = NCHUNKS // NC
PER_TILE = (CHUNKS_PER_SC // NS) & ~1
NBUF = 3
SEG_PER_SC = NSYS // NC
DUMMY = SEG_PER_SC
ACC_ROWS = SEG_PER_SC + 8
WB = 40
NWB = SEG_PER_SC // WB


def _sc_segment_sum(features, batch_index):
    mesh = plsc.VectorSubcoreMesh(core_axis_name="c", subcore_axis_name="s")

    @functools.partial(
        pl.kernel,
        out_type=jax.ShapeDtypeStruct((NSYS, D), jnp.float32),
        mesh=mesh,
        scratch_types=[
            pltpu.VMEM((NBUF, CHUNK, D), jnp.float32),
            pltpu.VMEM((NBUF, IROWS, 128), jnp.int32),
            pltpu.VMEM((WB, D), jnp.float32),
            pltpu.VMEM_SHARED((ACC_ROWS, D), jnp.float32),
            pltpu.SemaphoreType.DMA,
            pltpu.SemaphoreType.DMA,
            pltpu.SemaphoreType.DMA,
            pltpu.SemaphoreType.DMA,
            pltpu.SemaphoreType.DMA,
            pltpu.SemaphoreType.DMA,
        ],
    )
    def body(feat_hbm, idx_hbm, out_hbm, row_v, idx_v, zero_v, acc,
             g0, g1, g2, s0, s1, s2):
        c = lax.axis_index("c")
        t = lax.axis_index("s")
        segbase = SEG_PER_SC * c

        dummy_vec = jnp.full((16,), DUMMY, jnp.int32)

        def localize_idx(p, invalid_off):
            for q in range(IROWS):
                for k in range(128 // 16):
                    v = idx_v[p, q, pl.ds(16 * k, 16)] - segbase + invalid_off
                    m = (v >= 0) & (v < SEG_PER_SC)
                    idx_v[p, q, pl.ds(16 * k, 16)] = jnp.where(m, v, dummy_vec)

        base = CHUNKS_PER_SC * c
        gsem = (g0, g1, g2)
        ssem = (s0, s1, s2)

        def rstart(i):
            return CHUNK * (base + t + NS * i)

        def start_gather(i, p):
            r = rstart(i)
            for q in range(IROWS):
                pltpu.async_copy(
                    idx_hbm.at[pl.ds(r + 128 * q, 128)], idx_v.at[p, q], gsem[p]
                )
            pltpu.async_copy(feat_hbm.at[pl.ds(r, CHUNK)], row_v.at[p], gsem[p])

        def wait_gather(i, p):
            r = rstart(i)
            for q in range(IROWS):
                pltpu.make_async_copy(
                    idx_hbm.at[pl.ds(r + 128 * q, 128)], idx_v.at[p, q], gsem[p]
                ).wait()
            pltpu.make_async_copy(
                feat_hbm.at[pl.ds(r, CHUNK)], row_v.at[p], gsem[p]
            ).wait()

        def start_scatter(p):
            for q in range(IROWS):
                pltpu.async_copy(
                    row_v.at[p, pl.ds(128 * q, 128)],
                    acc.at[idx_v.at[p, q]],
                    ssem[p],
                    add=True,
                )

        def wait_scatter(p):
            for q in range(IROWS):
                pltpu.make_async_copy(
                    row_v.at[p, pl.ds(128 * q, 128)],
                    acc.at[idx_v.at[p, q]],
                    ssem[p],
                ).wait()

        def step(i, p, wait_prev, issue_gather):
            wait_gather(i, p)
            localize_idx(p, 0)
            if issue_gather:
                g = (p + 2) % NBUF
                if wait_prev:
                    wait_scatter(g)
                start_gather(i + 2, g)
            start_scatter(p)

        start_gather(0, 0)
        start_gather(1, 1)

        def zrow(i, _):
            for k in range(D // 16):
                zero_v[i, pl.ds(16 * k, 16)] = jnp.zeros((16,), jnp.float32)
            return 0

        lax.fori_loop(0, WB, zrow, 0)

        def zchunk(i, _):
            j = t + NS * i

            @pl.when(j < NWB)
            def _():
                pltpu.async_copy(zero_v, acc.at[pl.ds(WB * j, WB)], s0)

            return 0

        lax.fori_loop(0, (NWB + NS - 1) // NS, zchunk, 0)

        def zdrain(i, _):
            j = t + NS * i

            @pl.when(j < NWB)
            def _():
                pltpu.make_async_copy(
                    zero_v, acc.at[pl.ds(WB * j, WB)], s0
                ).wait()

            return 0

        lax.fori_loop(0, (NWB + NS - 1) // NS, zdrain, 0)
        plsc.subcore_barrier()

        step(0, 0, False, True)
        step(1, 1, True, True)
        step(2, 2, True, True)

        def aloop(i4, _):
            i = NBUF * i4
            for p in range(NBUF):
                step(i + p, p, True, True)
            return 0

        lax.fori_loop(1, PER_TILE // NBUF - 1, aloop, 0)

        step(PER_TILE - 3, (PER_TILE - 3) % NBUF, True, True)
        step(PER_TILE - 2, (PER_TILE - 2) % NBUF, False, False)
        step(PER_TILE - 1, (PER_TILE - 1) % NBUF, False, False)
        for p in range(NBUF):
            wait_scatter(p)

        n_left = CHUNKS_PER_SC - NS * PER_TILE
        for e in range((n_left + NS - 1) // NS):

            @pl.when(NS * e + t < n_left)
            def _():
                r = CHUNK * (base + NS * PER_TILE + NS * e + t)
                for q in range(IROWS):
                    pltpu.sync_copy(
                        idx_hbm.at[pl.ds(r + 128 * q, 128)], idx_v.at[0, q]
                    )
                pltpu.sync_copy(feat_hbm.at[pl.ds(r, CHUNK)], row_v.at[0])
                localize_idx(0, 0)
                for q in range(IROWS):
                    pltpu.sync_copy(
                        row_v.at[0, pl.ds(128 * q, 128)],
                        acc.at[idx_v.at[0, q]],
                        add=True,
                    )

        k0 = (1 - c) * (CHUNKS_PER_SC + t) + c * (CHUNKS_PER_SC - 1 - t)
        kstep = NS - 2 * NS * c

        max_sweep = (CHUNKS_PER_SC + NS - 1) // NS

        def sweep_body(i, cont):
            k = k0 + kstep * i
            kc = jnp.clip(k, 0, NCHUNKS - 1)
            ok = (cont == 1) & (k >= 0) & (k < NCHUNKS)

            @pl.when(ok)
            def _():
                for q in range(IROWS):
                    pltpu.sync_copy(
                        idx_hbm.at[pl.ds(CHUNK * kc + 128 * q, 128)],
                        idx_v.at[0, q],
                    )

            mn = idx_v[0, 0, pl.ds(0, 16)][0]
            mx = idx_v[0, IROWS - 1, pl.ds(112, 16)][15]
            a = (mn < SEG_PER_SC).astype(jnp.int32)
            b = (mx >= SEG_PER_SC).astype(jnp.int32)
            has_own = (1 - c) * a + c * b
            proceed = ok & (has_own == 1)

            @pl.when(proceed)
            def _():
                pltpu.sync_copy(
                    feat_hbm.at[pl.ds(CHUNK * kc, CHUNK)], row_v.at[0]
                )
                localize_idx(0, 0)
                for q in range(IROWS):
                    pltpu.sync_copy(
                        row_v.at[0, pl.ds(128 * q, 128)],
                        acc.at[idx_v.at[0, q]],
                        add=True,
                    )

            return proceed.astype(jnp.int32)

        lax.fori_loop(0, max_sweep, sweep_body, jnp.int32(1))
        plsc.subcore_barrier()

        def wchunk(i, _):
            j = t + NS * i

            @pl.when(j < NWB)
            def _():
                pltpu.sync_copy(
                    acc.at[pl.ds(WB * j, WB)],
                    out_hbm.at[pl.ds(SEG_PER_SC * c + WB * j, WB)],
                )

            return 0

        lax.fori_loop(0, (NWB + NS - 1) // NS, wchunk, 0)

    return body(features, batch_index)


def kernel(features, batch_index, natoms):
    del natoms
    bi = batch_index.astype(jnp.int32)
    return _sc_segment_sum(features, bi)


# speedup vs baseline: 1.0685x; 1.0115x over previous
import functools

import jax
import jax.numpy as jnp
from jax import lax
from jax.experimental import pallas as pl
from jax.experimental.pallas import tpu as pltpu
from jax.experimental.pallas import tpu_sc as plsc

N = 320000
D = 128
NSYS = 10000
NC = 2
NS = 16
CHUNK = 128
IROWS = CHUNK // 128
NCHUNKS = N // CHUNK
CHUNKS_PER_SC = NCHUNKS // NC
PER_TILE = (CHUNKS_PER_SC // NS) & ~1
NBUF = 3
SEG_PER_SC = NSYS // NC
DUMMY = SEG_PER_SC
ACC_ROWS = SEG_PER_SC + 8
WB = 40
NWB = SEG_PER_SC // WB


def _sc_segment_sum(features, batch_index):
    mesh = plsc.VectorSubcoreMesh(core_axis_name="c", subcore_axis_name="s")

    @functools.partial(
        pl.kernel,
        out_type=jax.ShapeDtypeStruct((NSYS, D), jnp.float32),
        mesh=mesh,
        scratch_types=[
            pltpu.VMEM((NBUF, CHUNK, D), jnp.float32),
            pltpu.VMEM((NBUF, IROWS, 128), jnp.int32),
            pltpu.VMEM((WB, D), jnp.float32),
            pltpu.VMEM_SHARED((ACC_ROWS, D), jnp.float32),
            pltpu.SemaphoreType.DMA,
            pltpu.SemaphoreType.DMA,
            pltpu.SemaphoreType.DMA,
            pltpu.SemaphoreType.DMA,
            pltpu.SemaphoreType.DMA,
            pltpu.SemaphoreType.DMA,
        ],
    )
    def body(feat_hbm, idx_hbm, out_hbm, row_v, idx_v, zero_v, acc,
             g0, g1, g2, s0, s1, s2):
        c = lax.axis_index("c")
        t = lax.axis_index("s")
        segbase = SEG_PER_SC * c

        dummy_vec = jnp.full((16,), DUMMY, jnp.int32)

        def localize_idx(p, invalid_off):
            for q in range(IROWS):
                for k in range(128 // 16):
                    v = idx_v[p, q, pl.ds(16 * k, 16)] - segbase + invalid_off
                    m = (v >= 0) & (v < SEG_PER_SC)
                    idx_v[p, q, pl.ds(16 * k, 16)] = jnp.where(m, v, dummy_vec)

        base = CHUNKS_PER_SC * c
        gsem = (g0, g1, g2)
        ssem = (s0, s1, s2)

        def rstart(i):
            return CHUNK * (base + t + NS * i)

        def start_gather(i, p):
            r = rstart(i)
            for q in range(IROWS):
                pltpu.async_copy(
                    idx_hbm.at[pl.ds(r + 128 * q, 128)], idx_v.at[p, q], gsem[p]
                )
            pltpu.async_copy(feat_hbm.at[pl.ds(r, CHUNK)], row_v.at[p], gsem[p])

        def wait_gather(i, p):
            r = rstart(i)
            for q in range(IROWS):
                pltpu.make_async_copy(
                    idx_hbm.at[pl.ds(r + 128 * q, 128)], idx_v.at[p, q], gsem[p]
                ).wait()
            pltpu.make_async_copy(
                feat_hbm.at[pl.ds(r, CHUNK)], row_v.at[p], gsem[p]
            ).wait()

        def start_scatter(p):
            for q in range(IROWS):
                pltpu.async_copy(
                    row_v.at[p, pl.ds(128 * q, 128)],
                    acc.at[idx_v.at[p, q]],
                    ssem[p],
                    add=True,
                )

        def wait_scatter(p):
            for q in range(IROWS):
                pltpu.make_async_copy(
                    row_v.at[p, pl.ds(128 * q, 128)],
                    acc.at[idx_v.at[p, q]],
                    ssem[p],
                ).wait()

        def step(i, p, wait_prev, issue_gather):
            wait_gather(i, p)
            localize_idx(p, 0)
            if issue_gather:
                g = (p + 2) % NBUF
                if wait_prev:
                    wait_scatter(g)
                start_gather(i + 2, g)
            start_scatter(p)

        start_gather(0, 0)
        start_gather(1, 1)

        def zrow(i, _):
            for k in range(D // 16):
                zero_v[i, pl.ds(16 * k, 16)] = jnp.zeros((16,), jnp.float32)
            return 0

        lax.fori_loop(0, WB, zrow, 0)

        def zchunk(i, _):
            j = t + NS * i

            @pl.when(j < NWB)
            def _():
                pltpu.async_copy(zero_v, acc.at[pl.ds(WB * j, WB)], s0)

            return 0

        lax.fori_loop(0, (NWB + NS - 1) // NS, zchunk, 0)

        def zdrain(i, _):
            j = t + NS * i

            @pl.when(j < NWB)
            def _():
                pltpu.make_async_copy(
                    zero_v, acc.at[pl.ds(WB * j, WB)], s0
                ).wait()

            return 0

        lax.fori_loop(0, (NWB + NS - 1) // NS, zdrain, 0)
        plsc.subcore_barrier()

        step(0, 0, False, True)
        step(1, 1, True, True)
        step(2, 2, True, True)

        def aloop(i4, _):
            i = NBUF * i4
            for p in range(NBUF):
                step(i + p, p, True, True)
            return 0

        lax.fori_loop(1, PER_TILE // NBUF - 1, aloop, 0)

        step(PER_TILE - 3, (PER_TILE - 3) % NBUF, True, True)
        step(PER_TILE - 2, (PER_TILE - 2) % NBUF, False, False)
        step(PER_TILE - 1, (PER_TILE - 1) % NBUF, False, False)
        for p in range(NBUF):
            wait_scatter(p)

        n_left = CHUNKS_PER_SC - NS * PER_TILE
        for e in range((n_left + NS - 1) // NS):

            @pl.when(NS * e + t < n_left)
            def _():
                r = CHUNK * (base + NS * PER_TILE + NS * e + t)
                for q in range(IROWS):
                    pltpu.sync_copy(
                        idx_hbm.at[pl.ds(r + 128 * q, 128)], idx_v.at[0, q]
                    )
                pltpu.sync_copy(feat_hbm.at[pl.ds(r, CHUNK)], row_v.at[0])
                localize_idx(0, 0)
                for q in range(IROWS):
                    pltpu.sync_copy(
                        row_v.at[0, pl.ds(128 * q, 128)],
                        acc.at[idx_v.at[0, q]],
                        add=True,
                    )

        k0 = (1 - c) * (CHUNKS_PER_SC + t) + c * (CHUNKS_PER_SC - 1 - t)
        kstep = NS - 2 * NS * c

        max_sweep = (CHUNKS_PER_SC + NS - 1) // NS

        def sweep_body(i, cont):
            k = k0 + kstep * i
            kc = jnp.clip(k, 0, NCHUNKS - 1)
            ok = (cont == 1) & (k >= 0) & (k < NCHUNKS)

            @pl.when(ok)
            def _():
                for q in range(IROWS):
                    pltpu.sync_copy(
                        idx_hbm.at[pl.ds(CHUNK * kc + 128 * q, 128)],
                        idx_v.at[0, q],
                    )

            mn = idx_v[0, 0, pl.ds(0, 16)][0]
            mx = idx_v[0, IROWS - 1, pl.ds(112, 16)][15]
            a = (mn < SEG_PER_SC).astype(jnp.int32)
            b = (mx >= SEG_PER_SC).astype(jnp.int32)
            has_own = (1 - c) * a + c * b
            proceed = ok & (has_own == 1)

            @pl.when(proceed)
            def _():
                pltpu.sync_copy(
                    feat_hbm.at[pl.ds(CHUNK * kc, CHUNK)], row_v.at[0]
                )
                localize_idx(0, 0)
                for q in range(IROWS):
                    pltpu.sync_copy(
                        row_v.at[0, pl.ds(128 * q, 128)],
                        acc.at[idx_v.at[0, q]],
                        add=True,
                    )

            return proceed.astype(jnp.int32)

        lax.fori_loop(0, max_sweep, sweep_body, jnp.int32(1))
        plsc.subcore_barrier()

        def wchunk(i, _):
            j = t + NS * i

            @pl.when(j < NWB)
            def _():
                pltpu.async_copy(
                    acc.at[pl.ds(WB * j, WB)],
                    out_hbm.at[pl.ds(SEG_PER_SC * c + WB * j, WB)],
                    s0,
                )

            return 0

        lax.fori_loop(0, (NWB + NS - 1) // NS, wchunk, 0)

        def wdrain(i, _):
            j = t + NS * i

            @pl.when(j < NWB)
            def _():
                pltpu.make_async_copy(
                    acc.at[pl.ds(WB * j, WB)],
                    out_hbm.at[pl.ds(SEG_PER_SC * c + WB * j, WB)],
                    s0,
                ).wait()

            return 0

        lax.fori_loop(0, (NWB + NS - 1) // NS, wdrain, 0)

    return body(features, batch_index)


def kernel(features, batch_index, natoms):
    del natoms
    bi = batch_index.astype(jnp.int32)
    return _sc_segment_sum(features, bi)


# speedup vs baseline: 1.0698x; 1.0012x over previous
import functools

import jax
import jax.numpy as jnp
from jax import lax
from jax.experimental import pallas as pl
from jax.experimental.pallas import tpu as pltpu
from jax.experimental.pallas import tpu_sc as plsc

N = 320000
D = 128
NSYS = 10000
NC = 2
NS = 16
CHUNK = 128
IROWS = CHUNK // 128
NCHUNKS = N // CHUNK
CHUNKS_PER_SC = NCHUNKS // NC
NBUF = 3
PER_TILE = (CHUNKS_PER_SC // NS) // NBUF * NBUF
SEG_PER_SC = NSYS // NC
DUMMY = SEG_PER_SC
ACC_ROWS = SEG_PER_SC + 8
WB = 40
NWB = SEG_PER_SC // WB


def _sc_segment_sum(features, batch_index):
    mesh = plsc.VectorSubcoreMesh(core_axis_name="c", subcore_axis_name="s")

    @functools.partial(
        pl.kernel,
        out_type=jax.ShapeDtypeStruct((NSYS, D), jnp.float32),
        mesh=mesh,
        scratch_types=[
            pltpu.VMEM((NBUF, CHUNK, D), jnp.float32),
            pltpu.VMEM((NBUF, IROWS, 128), jnp.int32),
            pltpu.VMEM((WB, D), jnp.float32),
            pltpu.VMEM_SHARED((ACC_ROWS, D), jnp.float32),
            pltpu.SemaphoreType.DMA,
            pltpu.SemaphoreType.DMA,
            pltpu.SemaphoreType.DMA,
            pltpu.SemaphoreType.DMA,
            pltpu.SemaphoreType.DMA,
            pltpu.SemaphoreType.DMA,
        ],
    )
    def body(feat_hbm, idx_hbm, out_hbm, row_v, idx_v, zero_v, acc,
             g0, g1, g2, s0, s1, s2):
        c = lax.axis_index("c")
        t = lax.axis_index("s")
        segbase = SEG_PER_SC * c

        dummy_vec = jnp.full((16,), DUMMY, jnp.int32)

        def localize_idx(p, invalid_off):
            for q in range(IROWS):
                for k in range(128 // 16):
                    v = idx_v[p, q, pl.ds(16 * k, 16)] - segbase + invalid_off
                    m = (v >= 0) & (v < SEG_PER_SC)
                    idx_v[p, q, pl.ds(16 * k, 16)] = jnp.where(m, v, dummy_vec)

        base = CHUNKS_PER_SC * c
        gsem = (g0, g1, g2)
        ssem = (s0, s1, s2)

        def rstart(i):
            return CHUNK * (base + t + NS * i)

        def start_gather(i, p):
            r = rstart(i)
            for q in range(IROWS):
                pltpu.async_copy(
                    idx_hbm.at[pl.ds(r + 128 * q, 128)], idx_v.at[p, q], gsem[p]
                )
            pltpu.async_copy(feat_hbm.at[pl.ds(r, CHUNK)], row_v.at[p], gsem[p])

        def wait_gather(i, p):
            r = rstart(i)
            for q in range(IROWS):
                pltpu.make_async_copy(
                    idx_hbm.at[pl.ds(r + 128 * q, 128)], idx_v.at[p, q], gsem[p]
                ).wait()
            pltpu.make_async_copy(
                feat_hbm.at[pl.ds(r, CHUNK)], row_v.at[p], gsem[p]
            ).wait()

        def start_scatter(p):
            for q in range(IROWS):
                pltpu.async_copy(
                    row_v.at[p, pl.ds(128 * q, 128)],
                    acc.at[idx_v.at[p, q]],
                    ssem[p],
                    add=True,
                )

        def wait_scatter(p):
            for q in range(IROWS):
                pltpu.make_async_copy(
                    row_v.at[p, pl.ds(128 * q, 128)],
                    acc.at[idx_v.at[p, q]],
                    ssem[p],
                ).wait()

        def step(i, p, wait_prev, issue_gather):
            wait_gather(i, p)
            localize_idx(p, 0)
            if issue_gather:
                g = (p + 2) % NBUF
                if wait_prev:
                    wait_scatter(g)
                start_gather(i + 2, g)
            start_scatter(p)

        start_gather(0, 0)
        start_gather(1, 1)

        def zrow(i, _):
            for k in range(D // 16):
                zero_v[i, pl.ds(16 * k, 16)] = jnp.zeros((16,), jnp.float32)
            return 0

        lax.fori_loop(0, WB, zrow, 0)

        def zchunk(i, _):
            j = t + NS * i

            @pl.when(j < NWB)
            def _():
                pltpu.async_copy(zero_v, acc.at[pl.ds(WB * j, WB)], s0)

            return 0

        lax.fori_loop(0, (NWB + NS - 1) // NS, zchunk, 0)

        def zdrain(i, _):
            j = t + NS * i

            @pl.when(j < NWB)
            def _():
                pltpu.make_async_copy(
                    zero_v, acc.at[pl.ds(WB * j, WB)], s0
                ).wait()

            return 0

        lax.fori_loop(0, (NWB + NS - 1) // NS, zdrain, 0)
        plsc.subcore_barrier()

        step(0, 0, False, True)
        step(1, 1, True, True)
        step(2, 2, True, True)

        def aloop(i4, _):
            i = NBUF * i4
            for p in range(NBUF):
                step(i + p, p, True, True)
            return 0

        lax.fori_loop(1, PER_TILE // NBUF - 1, aloop, 0)

        step(PER_TILE - 3, (PER_TILE - 3) % NBUF, True, True)
        step(PER_TILE - 2, (PER_TILE - 2) % NBUF, False, False)
        step(PER_TILE - 1, (PER_TILE - 1) % NBUF, False, False)
        for p in range(NBUF):
            wait_scatter(p)

        n_left = CHUNKS_PER_SC - NS * PER_TILE
        for e in range((n_left + NS - 1) // NS):

            @pl.when(NS * e + t < n_left)
            def _():
                r = CHUNK * (base + NS * PER_TILE + NS * e + t)
                for q in range(IROWS):
                    pltpu.sync_copy(
                        idx_hbm.at[pl.ds(r + 128 * q, 128)], idx_v.at[0, q]
                    )
                pltpu.sync_copy(feat_hbm.at[pl.ds(r, CHUNK)], row_v.at[0])
                localize_idx(0, 0)
                for q in range(IROWS):
                    pltpu.sync_copy(
                        row_v.at[0, pl.ds(128 * q, 128)],
                        acc.at[idx_v.at[0, q]],
                        add=True,
                    )

        k0 = (1 - c) * (CHUNKS_PER_SC + t) + c * (CHUNKS_PER_SC - 1 - t)
        kstep = NS - 2 * NS * c

        max_sweep = (CHUNKS_PER_SC + NS - 1) // NS

        def sweep_body(i, cont):
            k = k0 + kstep * i
            kc = jnp.clip(k, 0, NCHUNKS - 1)
            ok = (cont == 1) & (k >= 0) & (k < NCHUNKS)

            @pl.when(ok)
            def _():
                for q in range(IROWS):
                    pltpu.sync_copy(
                        idx_hbm.at[pl.ds(CHUNK * kc + 128 * q, 128)],
                        idx_v.at[0, q],
                    )

            mn = idx_v[0, 0, pl.ds(0, 16)][0]
            mx = idx_v[0, IROWS - 1, pl.ds(112, 16)][15]
            a = (mn < SEG_PER_SC).astype(jnp.int32)
            b = (mx >= SEG_PER_SC).astype(jnp.int32)
            has_own = (1 - c) * a + c * b
            proceed = ok & (has_own == 1)

            @pl.when(proceed)
            def _():
                pltpu.sync_copy(
                    feat_hbm.at[pl.ds(CHUNK * kc, CHUNK)], row_v.at[0]
                )
                localize_idx(0, 0)
                for q in range(IROWS):
                    pltpu.sync_copy(
                        row_v.at[0, pl.ds(128 * q, 128)],
                        acc.at[idx_v.at[0, q]],
                        add=True,
                    )

            return proceed.astype(jnp.int32)

        lax.fori_loop(0, max_sweep, sweep_body, jnp.int32(1))
        plsc.subcore_barrier()

        def wchunk(i, _):
            j = t + NS * i

            @pl.when(j < NWB)
            def _():
                pltpu.async_copy(
                    acc.at[pl.ds(WB * j, WB)],
                    out_hbm.at[pl.ds(SEG_PER_SC * c + WB * j, WB)],
                    s0,
                )

            return 0

        lax.fori_loop(0, (NWB + NS - 1) // NS, wchunk, 0)

        def wdrain(i, _):
            j = t + NS * i

            @pl.when(j < NWB)
            def _():
                pltpu.make_async_copy(
                    acc.at[pl.ds(WB * j, WB)],
                    out_hbm.at[pl.ds(SEG_PER_SC * c + WB * j, WB)],
                    s0,
                ).wait()

            return 0

        lax.fori_loop(0, (NWB + NS - 1) // NS, wdrain, 0)

    return body(features, batch_index)


def kernel(features, batch_index, natoms):
    del natoms
    bi = batch_index.astype(jnp.int32)
    return _sc_segment_sum(features, bi)
